# baseline (speedup 1.0000x reference)
import functools

import jax
import jax.numpy as jnp
from jax import lax
from jax.experimental import pallas as pl
from jax.experimental.pallas import tpu as pltpu
from jax.experimental.pallas import tpu_sc as plsc

BATCH = 16384
N_FIELDS = 26
EMBEDDING_DIM = 64

_B = BATCH * N_FIELDS
_NC = 2
_NS = 16
_NW = _NC * _NS
_B_PER_W = BATCH // _NW
_BCHUNK = 128
_KB = _B_PER_W // _BCHUNK
_N_UNITS = N_FIELDS * _KB

_mesh = plsc.VectorSubcoreMesh(core_axis_name="c", subcore_axis_name="s")


@functools.partial(
    pl.kernel,
    mesh=_mesh,
    out_type=jax.ShapeDtypeStruct((N_FIELDS, EMBEDDING_DIM, BATCH), jnp.float32),
    scratch_types=[
        pltpu.VMEM((_B_PER_W * N_FIELDS,), jnp.int32),
        pltpu.VMEM((4, _BCHUNK), jnp.int32),
        pltpu.VMEM((4, _BCHUNK, EMBEDDING_DIM), jnp.float32),
        pltpu.VMEM((2, EMBEDDING_DIM, _BCHUNK + 1), jnp.float32),
        pltpu.SemaphoreType.DMA((4,)),
        pltpu.SemaphoreType.DMA((2,)),
    ],
    compiler_params=pltpu.CompilerParams(
        use_tc_tiling_on_sc=False, needs_layout_passes=False
    ),
)
def _sc_gather_t(idx_hbm, table_hbm, out_hbm, idx_v, glist_v, rows_v, slab_v,
                 gsems, ssems):
    wid = lax.axis_index("s") * _NC + lax.axis_index("c")
    b_lo = wid * _B_PER_W
    lane = lax.iota(jnp.int32, 16)
    lane26 = lane * N_FIELDS

    pltpu.sync_copy(idx_hbm.at[pl.ds(b_lo * N_FIELDS, _B_PER_W * N_FIELDS)],
                    idx_v)

    def unit_fk(u):
        f = u // _KB
        kb = u - f * _KB
        return f, kb

    def build_glist(u, rb):
        f, kb = unit_fk(u)
        base = (kb * _BCHUNK) * N_FIELDS + f
        for j in range(8):
            pos = lane26 + (base + j * 16 * N_FIELDS)
            vals = plsc.load_gather(idx_v, [pos])
            glist_v[rb, pl.ds(j * 16, 16)] = vals

    def start_gather(rb):
        pltpu.async_copy(
            table_hbm.at[glist_v.at[rb]], rows_v.at[rb], gsems.at[rb]
        )

    def wait_gather(rb):
        pltpu.make_async_copy(
            table_hbm.at[glist_v.at[rb]], rows_v.at[rb], gsems.at[rb]
        ).wait()

    def start_store(u, rb):
        f, kb = unit_fk(u)
        pltpu.async_copy(
            slab_v.at[rb, :, pl.ds(0, _BCHUNK)],
            out_hbm.at[f, :, pl.ds(b_lo + kb * _BCHUNK, _BCHUNK)],
            ssems.at[rb],
        )

    def wait_store(rb):
        pltpu.make_async_copy(
            slab_v.at[rb, :, pl.ds(0, _BCHUNK)],
            out_hbm.at[0, :, pl.ds(0, _BCHUNK)], ssems.at[rb]
        ).wait()

    def transpose_chunk(grb, srb):
        for r in range(_BCHUNK):
            r_vec = jnp.full((16,), r, jnp.int32)
            for j in range(EMBEDDING_DIM // 16):
                val = rows_v[grb, r, pl.ds(j * 16, 16)]
                plsc.store_scatter(
                    slab_v.at[srb], [lane + (j * 16), r_vec], val
                )

    for k in range(4):
        build_glist(k, k)
        start_gather(k)

    def unit_quad(uq, carry):
        for sub in range(4):
            u = uq * 4 + sub
            grb = sub
            srb = sub % 2
            wait_gather(grb)

            @pl.when(u >= 2)
            def _():
                wait_store(srb)

            transpose_chunk(grb, srb)
            start_store(u, srb)

            @pl.when(u + 4 < _N_UNITS)
            def _():
                build_glist(u + 4, grb)
                start_gather(grb)
        return carry

    lax.fori_loop(0, _N_UNITS // 4, unit_quad, 0)
    wait_store(0)
    wait_store(1)


def kernel(token_ids, weight):
    idx_flat = jnp.reshape(token_ids, (_B,)).astype(jnp.int32)
    out_t = _sc_gather_t(idx_flat, weight)
    return jnp.transpose(out_t, (2, 0, 1))


# speedup vs baseline: 1.0453x; 1.0453x over previous
import functools

import jax
import jax.numpy as jnp
from jax import lax
from jax.experimental import pallas as pl
from jax.experimental.pallas import tpu as pltpu
from jax.experimental.pallas import tpu_sc as plsc

BATCH = 16384
N_FIELDS = 26
EMBEDDING_DIM = 64

_B = BATCH * N_FIELDS
_NC = 2
_NS = 16
_NW = _NC * _NS
_B_PER_W = _B // _NW
_CHUNK = 128
_N_CHUNKS = _B_PER_W // _CHUNK
_NBUF = 8
_NGROUPS = _N_CHUNKS // _NBUF

_mesh = plsc.VectorSubcoreMesh(core_axis_name="c", subcore_axis_name="s")


@functools.partial(
    pl.kernel,
    mesh=_mesh,
    out_type=jax.ShapeDtypeStruct((_B, EMBEDDING_DIM), jnp.float32),
    scratch_types=[
        pltpu.VMEM((_B_PER_W,), jnp.int32),
        pltpu.VMEM((_NBUF, _CHUNK, EMBEDDING_DIM), jnp.float32),
        pltpu.SemaphoreType.DMA((_NBUF,)),
        pltpu.SemaphoreType.DMA((_NBUF,)),
    ],
    compiler_params=pltpu.CompilerParams(use_tc_tiling_on_sc=False),
)
def _sc_gather(idx_hbm, table_hbm, out_hbm, idx_v, rows_v, gsems, ssems):
    wid = lax.axis_index("s") * _NC + lax.axis_index("c")
    base = wid * _B_PER_W
    pltpu.sync_copy(idx_hbm.at[pl.ds(base, _B_PER_W)], idx_v)

    def start_gather(chunk, b):
        pltpu.async_copy(
            table_hbm.at[idx_v.at[pl.ds(chunk * _CHUNK, _CHUNK)]],
            rows_v.at[b],
            gsems.at[b],
        )

    def wait_gather(b):
        pltpu.make_async_copy(
            table_hbm.at[idx_v.at[pl.ds(0, _CHUNK)]], rows_v.at[b], gsems.at[b]
        ).wait()

    def start_store(chunk, b):
        pltpu.async_copy(
            rows_v.at[b], out_hbm.at[pl.ds(base + chunk * _CHUNK, _CHUNK)],
            ssems.at[b],
        )

    def wait_store(b):
        pltpu.make_async_copy(
            rows_v.at[b], out_hbm.at[pl.ds(base, _CHUNK)], ssems.at[b]
        ).wait()

    for b in range(_NBUF):
        start_gather(b, b)

    def group_body(g, carry):
        for b in range(_NBUF):
            wait_gather(b)
            start_store(g * _NBUF + b, b)
        for b in range(_NBUF):

            @pl.when(g + 1 < _NGROUPS)
            def _():
                wait_store(b)
                start_gather((g + 1) * _NBUF + b, b)

        return carry

    lax.fori_loop(0, _NGROUPS, group_body, 0)

    for b in range(_NBUF):
        wait_store(b)


def kernel(token_ids, weight):
    idx_flat = jnp.reshape(token_ids, (_B,)).astype(jnp.int32)
    out = _sc_gather(idx_flat, weight)
    return jnp.reshape(out, (BATCH, N_FIELDS, EMBEDDING_DIM))
